# Initial kernel scaffold; baseline (speedup 1.0000x reference)
import jax
import jax.numpy as jnp
from jax.experimental import pallas as pl


def kernel(x, edge_index, B_fourier, W_enc1, b_enc1, W_enc2, b_enc2, sage_Wl, sage_bl, sage_Wr, ln_g, ln_b, W_h1, b_h1, W_h2, b_h2):
    raise NotImplementedError("write your pallas kernel here")


# speedup vs baseline: 4.6199x; 4.6199x over previous
import functools
import math

import jax
import jax.numpy as jnp
from jax import lax
from jax.experimental import pallas as pl
from jax.experimental.pallas import tpu as pltpu
from jax.experimental.pallas import tpu_sc as plsc

N = 10000
E = 320000
H = 128
M = 64
L = 4

CHUNK = 64
TILES = 32
CPT = 160
GC = 8
NG = CPT // GC
EPAD = CHUNK * TILES * CPT
NCHUNK = EPAD // CHUNK
NACC = 10112
ZROWS = NACC // 16

_SLICES = [(o, min(CHUNK, ZROWS - o)) for o in range(0, ZROWS, CHUNK)]

_f32 = jnp.float32


def _mesh():
    return plsc.VectorSubcoreMesh(core_axis_name="c", subcore_axis_name="s",
                                  num_cores=2, num_subcores=16)


def _make_agg():
    def body(h_hbm, src_hbm, dst_hbm, z128_hbm, aggr_out, acc, sidx, didx,
             rows):
        cid = lax.axis_index("c")
        sid = lax.axis_index("s")
        wid = cid * 16 + sid

        t0 = pl.multiple_of(sid * ZROWS, 8)
        pltpu.sync_copy(z128_hbm, rows)
        for (o, sz) in _SLICES:
            pltpu.sync_copy(rows.at[pl.ds(0, sz)], acc.at[pl.ds(t0 + o, sz)])

        plsc.subcore_barrier()

        @pl.loop(0, NG)
        def _(g):
            c0 = pl.multiple_of(wid * CPT + g * GC, 8)
            pltpu.sync_copy(src_hbm.at[pl.ds(c0, GC)], sidx)
            pltpu.sync_copy(dst_hbm.at[pl.ds(c0, GC)], didx)

            @pl.loop(0, GC)
            def _(j):
                pltpu.sync_copy(h_hbm.at[sidx.at[j]], rows)
                pltpu.sync_copy(rows, acc.at[didx.at[j]], add=True)

        plsc.subcore_barrier()

        for (o, sz) in _SLICES:
            pltpu.sync_copy(acc.at[pl.ds(t0 + o, sz)], rows.at[pl.ds(0, sz)])
            pltpu.sync_copy(rows.at[pl.ds(0, sz)],
                            aggr_out.at[cid, pl.ds(t0 + o, sz)])

    return pl.kernel(
        body,
        out_type=jax.ShapeDtypeStruct((2, NACC, H), _f32),
        mesh=_mesh(),
        scratch_types=[
            pltpu.VMEM_SHARED((NACC, H), _f32),
            pltpu.VMEM((GC, CHUNK), jnp.int32),
            pltpu.VMEM((GC, CHUNK), jnp.int32),
            pltpu.VMEM((CHUNK, H), _f32),
        ])


def _make_deg():
    def body(dst_hbm, z128_hbm, ones_hbm, deg_out, dacc, didx, rows):
        cid = lax.axis_index("c")
        sid = lax.axis_index("s")
        wid = cid * 16 + sid

        t0 = pl.multiple_of(sid * ZROWS, 8)
        pltpu.sync_copy(z128_hbm, rows)
        for (o, sz) in _SLICES:
            pltpu.sync_copy(rows.at[pl.ds(0, sz)], dacc.at[pl.ds(t0 + o, sz)])
        pltpu.sync_copy(ones_hbm, rows)

        plsc.subcore_barrier()

        @pl.loop(0, NG)
        def _(g):
            c0 = pl.multiple_of(wid * CPT + g * GC, 8)
            pltpu.sync_copy(dst_hbm.at[pl.ds(c0, GC)], didx)

            @pl.loop(0, GC)
            def _(j):
                pltpu.sync_copy(rows, dacc.at[didx.at[j]], add=True)

        plsc.subcore_barrier()

        for (o, sz) in _SLICES:
            pltpu.sync_copy(dacc.at[pl.ds(t0 + o, sz)], rows.at[pl.ds(0, sz)])
            pltpu.sync_copy(rows.at[pl.ds(0, sz)],
                            deg_out.at[cid, pl.ds(t0 + o, sz)])

    return pl.kernel(
        body,
        out_type=jax.ShapeDtypeStruct((2, NACC, H), _f32),
        mesh=_mesh(),
        scratch_types=[
            pltpu.VMEM_SHARED((NACC, H), _f32),
            pltpu.VMEM((GC, CHUNK), jnp.int32),
            pltpu.VMEM((CHUNK, H), _f32),
        ])


@functools.cache
def _get_agg():
    return _make_agg()


@functools.cache
def _get_deg():
    return _make_deg()


BN = 1000
GRID = N // BN

_TWO_PI = 2.0 * math.pi


def _silu(v):
    return v * (1.0 / (1.0 + jnp.exp(-v)))


def _dot(a, b):
    return jax.lax.dot_general(a, b, (((1,), (0,)), ((), ())),
                               precision=jax.lax.Precision.HIGHEST,
                               preferred_element_type=_f32)


def _row_spec(width):
    return pl.BlockSpec((BN, width), lambda i: (i, 0))


def _full_spec(shape):
    nd = len(shape)
    return pl.BlockSpec(shape, lambda i: (0,) * nd)


def _sincos_2pi(t):
    r = t - jnp.round(t)
    q = jnp.round(4.0 * r)
    u = (r - 0.25 * q) * _TWO_PI
    u2 = u * u
    su = u * (1.0 + u2 * (-1.0 / 6.0 + u2 * (1.0 / 120.0
              + u2 * (-1.0 / 5040.0 + u2 * (1.0 / 362880.0)))))
    cu = 1.0 + u2 * (-0.5 + u2 * (1.0 / 24.0 + u2 * (-1.0 / 720.0
              + u2 * (1.0 / 40320.0))))
    m = q - 4.0 * jnp.floor(q * 0.25)
    sin_q = jnp.where(m == 1.0, 1.0, jnp.where(m == 3.0, -1.0, 0.0))
    cos_q = jnp.where(m == 0.0, 1.0, jnp.where(m == 2.0, -1.0, 0.0))
    return su * cos_q + cu * sin_q, cu * cos_q - su * sin_q


def _enc_body(x_ref, sin_ref, cos_ref, w1x_ref, w1c_ref, w1s_ref, w1k_ref,
              b1_ref, w2_ref, b2_ref, h_ref):
    xb = x_ref[...]
    w1c = w1c_ref[...]
    pre = (_dot(xb, w1x_ref[...])
           + xb[:, 0:1] * w1c[0:1, :] + xb[:, 1:2] * w1c[1:2, :]
           + xb[:, 2:3] * w1c[2:3, :]
           + _dot(sin_ref[...], w1s_ref[...])
           + _dot(cos_ref[...], w1k_ref[...])
           + b1_ref[...])
    h_ref[...] = _dot(_silu(pre), w2_ref[...]) + b2_ref[...]


def _encoder(x, sin_p, cos_p, W_enc1, b_enc1, W_enc2, b_enc2):
    return pl.pallas_call(
        _enc_body,
        grid=(GRID,),
        in_specs=[
            _row_spec(128), _row_spec(M), _row_spec(M),
            _full_spec((128, H)), _full_spec((3, H)),
            _full_spec((M, H)), _full_spec((M, H)), _full_spec((1, H)),
            _full_spec((H, H)), _full_spec((1, H)),
        ],
        out_specs=_row_spec(H),
        out_shape=jax.ShapeDtypeStruct((N, H), _f32),
    )(x, sin_p, cos_p, W_enc1[:128], W_enc1[128:131], W_enc1[131:195],
      W_enc1[195:259], b_enc1.reshape(1, H), W_enc2, b_enc2.reshape(1, H))


def _layer_body(h_ref, a0_ref, a1_ref, d0_ref, d1_ref, wl_ref, wr_ref,
                bl_ref, g_ref, b_ref, out_ref):
    h = h_ref[...]
    aggr = a0_ref[...] + a1_ref[...]
    deg = d0_ref[...][:, 0:1] + d1_ref[...][:, 0:1]
    scale = 1.0 / jnp.maximum(deg, 1.0)
    out = _dot(aggr * scale, wl_ref[...]) + _dot(h, wr_ref[...]) + bl_ref[...]
    nrm = jnp.sqrt(jnp.sum(out * out, axis=-1, keepdims=True))
    out = out / jnp.maximum(nrm, 1e-12)
    mu = jnp.mean(out, axis=-1, keepdims=True)
    cen = out - mu
    var = jnp.mean(cen * cen, axis=-1, keepdims=True)
    ln = cen / jnp.sqrt(var + 1e-5) * g_ref[...] + b_ref[...]
    out_ref[...] = _silu(ln) + h


def _layer(h, a0, a1, d0, d1, wl, bl, wr, g, b):
    return pl.pallas_call(
        _layer_body,
        grid=(GRID,),
        in_specs=[
            _row_spec(H), _row_spec(H), _row_spec(H),
            _row_spec(16), _row_spec(16),
            _full_spec((H, H)), _full_spec((H, H)),
            _full_spec((1, H)), _full_spec((1, H)), _full_spec((1, H)),
        ],
        out_specs=_row_spec(H),
        out_shape=jax.ShapeDtypeStruct((N, H), _f32),
    )(h, a0, a1, d0, d1, wl, wr, bl.reshape(1, H), g.reshape(1, H),
      b.reshape(1, H))


def _head_body(h_ref, w1_ref, b1_ref, w2_ref, b2_ref, out_ref):
    t = _silu(_dot(h_ref[...], w1_ref[...]) + b1_ref[...])
    out_ref[...] = _dot(t, w2_ref[...]) + b2_ref[...]


def _head(h, W_h1, b_h1, W_h2, b_h2):
    return pl.pallas_call(
        _head_body,
        grid=(GRID,),
        in_specs=[
            _row_spec(H), _full_spec((H, H)), _full_spec((1, H)),
            _full_spec((H, 3)), _full_spec((1, 3)),
        ],
        out_specs=_row_spec(3),
        out_shape=jax.ShapeDtypeStruct((N, 3), _f32),
    )(h, W_h1, b_h1.reshape(1, H), W_h2, b_h2.reshape(1, 3))


def kernel(x, edge_index, B_fourier, W_enc1, b_enc1, W_enc2, b_enc2,
           sage_Wl, sage_bl, sage_Wr, ln_g, ln_b, W_h1, b_h1, W_h2, b_h2):
    src = edge_index[0].astype(jnp.int32)
    dst = edge_index[1].astype(jnp.int32)
    pad = EPAD - E
    ar = jnp.arange(pad, dtype=jnp.int32)
    src_p = jnp.concatenate([src, (ar * 37) % N]).reshape(NCHUNK, CHUNK)
    dst_p = jnp.concatenate([dst, N + (ar % (NACC - N))]).reshape(NCHUNK, CHUNK)
    z128 = jnp.zeros((CHUNK, H), _f32)
    ones128 = jnp.ones((CHUNK, H), _f32)

    proj = _TWO_PI * (x[:, :3] @ B_fourier)
    h = _encoder(x, jnp.sin(proj), jnp.cos(proj), W_enc1, b_enc1, W_enc2,
                 b_enc2)

    degs = _get_deg()(dst_p, z128, ones128)
    d0 = degs[0, :, :16]
    d1 = degs[1, :, :16]
    for i in range(L):
        aggr = _get_agg()(h, src_p, dst_p, z128)
        h = _layer(h, aggr[0], aggr[1], d0, d1,
                   sage_Wl[i], sage_bl[i], sage_Wr[i], ln_g[i], ln_b[i])

    return _head(h, W_h1, b_h1, W_h2, b_h2)


# speedup vs baseline: 5.7743x; 1.2499x over previous
import functools
import math

import jax
import jax.numpy as jnp
from jax import lax
from jax.experimental import pallas as pl
from jax.experimental.pallas import tpu as pltpu
from jax.experimental.pallas import tpu_sc as plsc

N = 10000
E = 320000
H = 128
M = 64
L = 4

CHUNK = 128
TILES = 32
CPT = 80
GC = 8
NG = CPT // GC
EPAD = CHUNK * TILES * CPT
NCHUNK = EPAD // CHUNK
NACC = 10112
ZROWS = NACC // 16

_SLICES = [(o, min(CHUNK, ZROWS - o)) for o in range(0, ZROWS, CHUNK)]

_f32 = jnp.float32


def _mesh():
    return plsc.VectorSubcoreMesh(core_axis_name="c", subcore_axis_name="s",
                                  num_cores=2, num_subcores=16)


def _make_agg():
    def body(h_hbm, src_hbm, dst_hbm, z128_hbm, aggr_out, acc, sidx, didx,
             rows):
        cid = lax.axis_index("c")
        sid = lax.axis_index("s")
        wid = cid * 16 + sid

        t0 = pl.multiple_of(sid * ZROWS, 8)
        pltpu.sync_copy(z128_hbm, rows)
        for (o, sz) in _SLICES:
            pltpu.sync_copy(rows.at[pl.ds(0, sz)], acc.at[pl.ds(t0 + o, sz)])

        plsc.subcore_barrier()

        @pl.loop(0, NG)
        def _(g):
            c0 = pl.multiple_of(wid * CPT + g * GC, 8)
            pltpu.sync_copy(src_hbm.at[pl.ds(c0, GC)], sidx)
            pltpu.sync_copy(dst_hbm.at[pl.ds(c0, GC)], didx)

            @pl.loop(0, GC)
            def _(j):
                pltpu.sync_copy(h_hbm.at[sidx.at[j]], rows)
                pltpu.sync_copy(rows, acc.at[didx.at[j]], add=True)

        plsc.subcore_barrier()

        for (o, sz) in _SLICES:
            pltpu.sync_copy(acc.at[pl.ds(t0 + o, sz)], rows.at[pl.ds(0, sz)])
            pltpu.sync_copy(rows.at[pl.ds(0, sz)],
                            aggr_out.at[cid, pl.ds(t0 + o, sz)])

    return pl.kernel(
        body,
        out_type=jax.ShapeDtypeStruct((2, NACC, H), _f32),
        mesh=_mesh(),
        scratch_types=[
            pltpu.VMEM_SHARED((NACC, H), _f32),
            pltpu.VMEM((GC, CHUNK), jnp.int32),
            pltpu.VMEM((GC, CHUNK), jnp.int32),
            pltpu.VMEM((CHUNK, H), _f32),
        ])


def _make_deg():
    def body(dst_hbm, z128_hbm, ones_hbm, deg_out, dacc, didx, rows):
        cid = lax.axis_index("c")
        sid = lax.axis_index("s")
        wid = cid * 16 + sid

        t0 = pl.multiple_of(sid * ZROWS, 8)
        pltpu.sync_copy(z128_hbm, rows)
        for (o, sz) in _SLICES:
            pltpu.sync_copy(rows.at[pl.ds(0, sz)], dacc.at[pl.ds(t0 + o, sz)])
        pltpu.sync_copy(ones_hbm, rows)

        plsc.subcore_barrier()

        @pl.loop(0, NG)
        def _(g):
            c0 = pl.multiple_of(wid * CPT + g * GC, 8)
            pltpu.sync_copy(dst_hbm.at[pl.ds(c0, GC)], didx)

            @pl.loop(0, GC)
            def _(j):
                pltpu.sync_copy(rows, dacc.at[didx.at[j]], add=True)

        plsc.subcore_barrier()

        for (o, sz) in _SLICES:
            pltpu.sync_copy(dacc.at[pl.ds(t0 + o, sz)], rows.at[pl.ds(0, sz)])
            pltpu.sync_copy(rows.at[pl.ds(0, sz)],
                            deg_out.at[cid, pl.ds(t0 + o, sz)])

    return pl.kernel(
        body,
        out_type=jax.ShapeDtypeStruct((2, NACC, H), _f32),
        mesh=_mesh(),
        scratch_types=[
            pltpu.VMEM_SHARED((NACC, H), _f32),
            pltpu.VMEM((GC, CHUNK), jnp.int32),
            pltpu.VMEM((CHUNK, H), _f32),
        ])


@functools.cache
def _get_agg():
    return _make_agg()


@functools.cache
def _get_deg():
    return _make_deg()


BN = 1000
GRID = N // BN

_TWO_PI = 2.0 * math.pi


def _silu(v):
    return v * (1.0 / (1.0 + jnp.exp(-v)))


def _dot(a, b):
    return jax.lax.dot_general(a, b, (((1,), (0,)), ((), ())),
                               precision=jax.lax.Precision.HIGHEST,
                               preferred_element_type=_f32)


def _row_spec(width):
    return pl.BlockSpec((BN, width), lambda i: (i, 0))


def _full_spec(shape):
    nd = len(shape)
    return pl.BlockSpec(shape, lambda i: (0,) * nd)


def _sincos_2pi(t):
    r = t - jnp.round(t)
    q = jnp.round(4.0 * r)
    u = (r - 0.25 * q) * _TWO_PI
    u2 = u * u
    su = u * (1.0 + u2 * (-1.0 / 6.0 + u2 * (1.0 / 120.0
              + u2 * (-1.0 / 5040.0 + u2 * (1.0 / 362880.0)))))
    cu = 1.0 + u2 * (-0.5 + u2 * (1.0 / 24.0 + u2 * (-1.0 / 720.0
              + u2 * (1.0 / 40320.0))))
    m = q - 4.0 * jnp.floor(q * 0.25)
    sin_q = jnp.where(m == 1.0, 1.0, jnp.where(m == 3.0, -1.0, 0.0))
    cos_q = jnp.where(m == 0.0, 1.0, jnp.where(m == 2.0, -1.0, 0.0))
    return su * cos_q + cu * sin_q, cu * cos_q - su * sin_q


def _enc_body(x_ref, sin_ref, cos_ref, w1x_ref, w1c_ref, w1s_ref, w1k_ref,
              b1_ref, w2_ref, b2_ref, h_ref):
    xb = x_ref[...]
    w1c = w1c_ref[...]
    pre = (_dot(xb, w1x_ref[...])
           + xb[:, 0:1] * w1c[0:1, :] + xb[:, 1:2] * w1c[1:2, :]
           + xb[:, 2:3] * w1c[2:3, :]
           + _dot(sin_ref[...], w1s_ref[...])
           + _dot(cos_ref[...], w1k_ref[...])
           + b1_ref[...])
    h_ref[...] = _dot(_silu(pre), w2_ref[...]) + b2_ref[...]


def _encoder(x, sin_p, cos_p, W_enc1, b_enc1, W_enc2, b_enc2):
    return pl.pallas_call(
        _enc_body,
        grid=(GRID,),
        in_specs=[
            _row_spec(128), _row_spec(M), _row_spec(M),
            _full_spec((128, H)), _full_spec((3, H)),
            _full_spec((M, H)), _full_spec((M, H)), _full_spec((1, H)),
            _full_spec((H, H)), _full_spec((1, H)),
        ],
        out_specs=_row_spec(H),
        out_shape=jax.ShapeDtypeStruct((N, H), _f32),
    )(x, sin_p, cos_p, W_enc1[:128], W_enc1[128:131], W_enc1[131:195],
      W_enc1[195:259], b_enc1.reshape(1, H), W_enc2, b_enc2.reshape(1, H))


def _layer_body(h_ref, a0_ref, a1_ref, d0_ref, d1_ref, wl_ref, wr_ref,
                bl_ref, g_ref, b_ref, out_ref):
    h = h_ref[...]
    aggr = a0_ref[...] + a1_ref[...]
    deg = d0_ref[...][:, 0:1] + d1_ref[...][:, 0:1]
    scale = 1.0 / jnp.maximum(deg, 1.0)
    out = _dot(aggr * scale, wl_ref[...]) + _dot(h, wr_ref[...]) + bl_ref[...]
    nrm = jnp.sqrt(jnp.sum(out * out, axis=-1, keepdims=True))
    out = out / jnp.maximum(nrm, 1e-12)
    mu = jnp.mean(out, axis=-1, keepdims=True)
    cen = out - mu
    var = jnp.mean(cen * cen, axis=-1, keepdims=True)
    ln = cen / jnp.sqrt(var + 1e-5) * g_ref[...] + b_ref[...]
    out_ref[...] = _silu(ln) + h


def _layer(h, a0, a1, d0, d1, wl, bl, wr, g, b):
    return pl.pallas_call(
        _layer_body,
        grid=(GRID,),
        in_specs=[
            _row_spec(H), _row_spec(H), _row_spec(H),
            _row_spec(16), _row_spec(16),
            _full_spec((H, H)), _full_spec((H, H)),
            _full_spec((1, H)), _full_spec((1, H)), _full_spec((1, H)),
        ],
        out_specs=_row_spec(H),
        out_shape=jax.ShapeDtypeStruct((N, H), _f32),
    )(h, a0, a1, d0, d1, wl, wr, bl.reshape(1, H), g.reshape(1, H),
      b.reshape(1, H))


def _head_body(h_ref, w1_ref, b1_ref, w2_ref, b2_ref, out_ref):
    t = _silu(_dot(h_ref[...], w1_ref[...]) + b1_ref[...])
    out_ref[...] = _dot(t, w2_ref[...]) + b2_ref[...]


def _head(h, W_h1, b_h1, W_h2, b_h2):
    return pl.pallas_call(
        _head_body,
        grid=(GRID,),
        in_specs=[
            _row_spec(H), _full_spec((H, H)), _full_spec((1, H)),
            _full_spec((H, 3)), _full_spec((1, 3)),
        ],
        out_specs=_row_spec(3),
        out_shape=jax.ShapeDtypeStruct((N, 3), _f32),
    )(h, W_h1, b_h1.reshape(1, H), W_h2, b_h2.reshape(1, 3))


def kernel(x, edge_index, B_fourier, W_enc1, b_enc1, W_enc2, b_enc2,
           sage_Wl, sage_bl, sage_Wr, ln_g, ln_b, W_h1, b_h1, W_h2, b_h2):
    src = edge_index[0].astype(jnp.int32)
    dst = edge_index[1].astype(jnp.int32)
    pad = EPAD - E
    ar = jnp.arange(pad, dtype=jnp.int32)
    src_p = jnp.concatenate([src, (ar * 37) % N]).reshape(NCHUNK, CHUNK)
    dst_p = jnp.concatenate([dst, N + (ar % (NACC - N))]).reshape(NCHUNK, CHUNK)
    z128 = jnp.zeros((CHUNK, H), _f32)
    ones128 = jnp.ones((CHUNK, H), _f32)

    proj = _TWO_PI * (x[:, :3] @ B_fourier)
    h = _encoder(x, jnp.sin(proj), jnp.cos(proj), W_enc1, b_enc1, W_enc2,
                 b_enc2)

    degs = _get_deg()(dst_p, z128, ones128)
    d0 = degs[0, :, :16]
    d1 = degs[1, :, :16]
    for i in range(L):
        aggr = _get_agg()(h, src_p, dst_p, z128)
        h = _layer(h, aggr[0], aggr[1], d0, d1,
                   sage_Wl[i], sage_bl[i], sage_Wr[i], ln_g[i], ln_b[i])

    return _head(h, W_h1, b_h1, W_h2, b_h2)


# speedup vs baseline: 6.7926x; 1.1763x over previous
import functools
import math

import jax
import jax.numpy as jnp
from jax import lax
from jax.experimental import pallas as pl
from jax.experimental.pallas import tpu as pltpu
from jax.experimental.pallas import tpu_sc as plsc

N = 10000
E = 320000
H = 128
M = 64
L = 4

CHUNK = 128
TILES = 32
CPT = 80
GC = 8
NG = CPT // GC
EPAD = CHUNK * TILES * CPT
NCHUNK = EPAD // CHUNK
NACC = 10112
ZROWS = NACC // 16

_SLICES = [(o, min(CHUNK, ZROWS - o)) for o in range(0, ZROWS, CHUNK)]

_f32 = jnp.float32


def _mesh():
    return plsc.VectorSubcoreMesh(core_axis_name="c", subcore_axis_name="s",
                                  num_cores=2, num_subcores=16)


def _make_agg():
    def body(h_hbm, src_hbm, dst_hbm, z128_hbm, aggr_out, acc, sidx, didx,
             rows, rows2, sem0, sem1):
        cid = lax.axis_index("c")
        sid = lax.axis_index("s")
        wid = cid * 16 + sid

        t0 = pl.multiple_of(sid * ZROWS, 8)
        pltpu.sync_copy(z128_hbm, rows)
        for (o, sz) in _SLICES:
            pltpu.sync_copy(rows.at[pl.ds(0, sz)], acc.at[pl.ds(t0 + o, sz)])

        plsc.subcore_barrier()

        bufs = (rows, rows2)
        sems = (sem0, sem1)

        @pl.loop(0, NG)
        def _(g):
            c0 = pl.multiple_of(wid * CPT + g * GC, 8)
            pltpu.sync_copy(src_hbm.at[pl.ds(c0, GC)], sidx)
            pltpu.sync_copy(dst_hbm.at[pl.ds(c0, GC)], didx)

            cps = [None, None]
            cps[0] = pltpu.async_copy(h_hbm.at[sidx.at[0]], bufs[0], sems[0])
            for j in range(GC):
                b = j % 2
                cps[b].wait()
                if j + 1 < GC:
                    nb = (j + 1) % 2
                    cps[nb] = pltpu.async_copy(h_hbm.at[sidx.at[j + 1]],
                                               bufs[nb], sems[nb])
                pltpu.sync_copy(bufs[b], acc.at[didx.at[j]], add=True)

        plsc.subcore_barrier()

        for (o, sz) in _SLICES:
            pltpu.sync_copy(acc.at[pl.ds(t0 + o, sz)], rows.at[pl.ds(0, sz)])
            pltpu.sync_copy(rows.at[pl.ds(0, sz)],
                            aggr_out.at[cid, pl.ds(t0 + o, sz)])

    return pl.kernel(
        body,
        out_type=jax.ShapeDtypeStruct((2, NACC, H), _f32),
        mesh=_mesh(),
        scratch_types=[
            pltpu.VMEM_SHARED((NACC, H), _f32),
            pltpu.VMEM((GC, CHUNK), jnp.int32),
            pltpu.VMEM((GC, CHUNK), jnp.int32),
            pltpu.VMEM((CHUNK, H), _f32),
            pltpu.VMEM((CHUNK, H), _f32),
            pltpu.SemaphoreType.DMA,
            pltpu.SemaphoreType.DMA,
        ])


def _make_deg():
    def body(dst_hbm, z128_hbm, ones_hbm, deg_out, dacc, didx, rows):
        cid = lax.axis_index("c")
        sid = lax.axis_index("s")
        wid = cid * 16 + sid

        t0 = pl.multiple_of(sid * ZROWS, 8)
        pltpu.sync_copy(z128_hbm, rows)
        for (o, sz) in _SLICES:
            pltpu.sync_copy(rows.at[pl.ds(0, sz)], dacc.at[pl.ds(t0 + o, sz)])
        pltpu.sync_copy(ones_hbm, rows)

        plsc.subcore_barrier()

        @pl.loop(0, NG)
        def _(g):
            c0 = pl.multiple_of(wid * CPT + g * GC, 8)
            pltpu.sync_copy(dst_hbm.at[pl.ds(c0, GC)], didx)

            @pl.loop(0, GC)
            def _(j):
                pltpu.sync_copy(rows, dacc.at[didx.at[j]], add=True)

        plsc.subcore_barrier()

        for (o, sz) in _SLICES:
            pltpu.sync_copy(dacc.at[pl.ds(t0 + o, sz)], rows.at[pl.ds(0, sz)])
            pltpu.sync_copy(rows.at[pl.ds(0, sz)],
                            deg_out.at[cid, pl.ds(t0 + o, sz)])

    return pl.kernel(
        body,
        out_type=jax.ShapeDtypeStruct((2, NACC, H), _f32),
        mesh=_mesh(),
        scratch_types=[
            pltpu.VMEM_SHARED((NACC, H), _f32),
            pltpu.VMEM((GC, CHUNK), jnp.int32),
            pltpu.VMEM((CHUNK, H), _f32),
        ])


@functools.cache
def _get_agg():
    return _make_agg()


@functools.cache
def _get_deg():
    return _make_deg()


BN = 1000
GRID = N // BN

_TWO_PI = 2.0 * math.pi


def _silu(v):
    return v * (1.0 / (1.0 + jnp.exp(-v)))


def _dot(a, b):
    return jax.lax.dot_general(a, b, (((1,), (0,)), ((), ())),
                               precision=jax.lax.Precision.HIGHEST,
                               preferred_element_type=_f32)


def _row_spec(width):
    return pl.BlockSpec((BN, width), lambda i: (i, 0))


def _full_spec(shape):
    nd = len(shape)
    return pl.BlockSpec(shape, lambda i: (0,) * nd)


def _sincos_2pi(t):
    r = t - jnp.round(t)
    q = jnp.round(4.0 * r)
    u = (r - 0.25 * q) * _TWO_PI
    u2 = u * u
    su = u * (1.0 + u2 * (-1.0 / 6.0 + u2 * (1.0 / 120.0
              + u2 * (-1.0 / 5040.0 + u2 * (1.0 / 362880.0)))))
    cu = 1.0 + u2 * (-0.5 + u2 * (1.0 / 24.0 + u2 * (-1.0 / 720.0
              + u2 * (1.0 / 40320.0))))
    m = q - 4.0 * jnp.floor(q * 0.25)
    sin_q = jnp.where(m == 1.0, 1.0, jnp.where(m == 3.0, -1.0, 0.0))
    cos_q = jnp.where(m == 0.0, 1.0, jnp.where(m == 2.0, -1.0, 0.0))
    return su * cos_q + cu * sin_q, cu * cos_q - su * sin_q


def _enc_body(x_ref, sin_ref, cos_ref, w1x_ref, w1c_ref, w1s_ref, w1k_ref,
              b1_ref, w2_ref, b2_ref, h_ref):
    xb = x_ref[...]
    w1c = w1c_ref[...]
    pre = (_dot(xb, w1x_ref[...])
           + xb[:, 0:1] * w1c[0:1, :] + xb[:, 1:2] * w1c[1:2, :]
           + xb[:, 2:3] * w1c[2:3, :]
           + _dot(sin_ref[...], w1s_ref[...])
           + _dot(cos_ref[...], w1k_ref[...])
           + b1_ref[...])
    h_ref[...] = _dot(_silu(pre), w2_ref[...]) + b2_ref[...]


def _encoder(x, sin_p, cos_p, W_enc1, b_enc1, W_enc2, b_enc2):
    return pl.pallas_call(
        _enc_body,
        grid=(GRID,),
        in_specs=[
            _row_spec(128), _row_spec(M), _row_spec(M),
            _full_spec((128, H)), _full_spec((3, H)),
            _full_spec((M, H)), _full_spec((M, H)), _full_spec((1, H)),
            _full_spec((H, H)), _full_spec((1, H)),
        ],
        out_specs=_row_spec(H),
        out_shape=jax.ShapeDtypeStruct((N, H), _f32),
    )(x, sin_p, cos_p, W_enc1[:128], W_enc1[128:131], W_enc1[131:195],
      W_enc1[195:259], b_enc1.reshape(1, H), W_enc2, b_enc2.reshape(1, H))


def _layer_body(h_ref, a0_ref, a1_ref, d0_ref, d1_ref, wl_ref, wr_ref,
                bl_ref, g_ref, b_ref, out_ref):
    h = h_ref[...]
    aggr = a0_ref[...] + a1_ref[...]
    deg = d0_ref[...][:, 0:1] + d1_ref[...][:, 0:1]
    scale = 1.0 / jnp.maximum(deg, 1.0)
    out = _dot(aggr * scale, wl_ref[...]) + _dot(h, wr_ref[...]) + bl_ref[...]
    nrm = jnp.sqrt(jnp.sum(out * out, axis=-1, keepdims=True))
    out = out / jnp.maximum(nrm, 1e-12)
    mu = jnp.mean(out, axis=-1, keepdims=True)
    cen = out - mu
    var = jnp.mean(cen * cen, axis=-1, keepdims=True)
    ln = cen / jnp.sqrt(var + 1e-5) * g_ref[...] + b_ref[...]
    out_ref[...] = _silu(ln) + h


def _layer(h, a0, a1, d0, d1, wl, bl, wr, g, b):
    return pl.pallas_call(
        _layer_body,
        grid=(GRID,),
        in_specs=[
            _row_spec(H), _row_spec(H), _row_spec(H),
            _row_spec(16), _row_spec(16),
            _full_spec((H, H)), _full_spec((H, H)),
            _full_spec((1, H)), _full_spec((1, H)), _full_spec((1, H)),
        ],
        out_specs=_row_spec(H),
        out_shape=jax.ShapeDtypeStruct((N, H), _f32),
    )(h, a0, a1, d0, d1, wl, wr, bl.reshape(1, H), g.reshape(1, H),
      b.reshape(1, H))


def _head_body(h_ref, w1_ref, b1_ref, w2_ref, b2_ref, out_ref):
    t = _silu(_dot(h_ref[...], w1_ref[...]) + b1_ref[...])
    out_ref[...] = _dot(t, w2_ref[...]) + b2_ref[...]


def _head(h, W_h1, b_h1, W_h2, b_h2):
    return pl.pallas_call(
        _head_body,
        grid=(GRID,),
        in_specs=[
            _row_spec(H), _full_spec((H, H)), _full_spec((1, H)),
            _full_spec((H, 3)), _full_spec((1, 3)),
        ],
        out_specs=_row_spec(3),
        out_shape=jax.ShapeDtypeStruct((N, 3), _f32),
    )(h, W_h1, b_h1.reshape(1, H), W_h2, b_h2.reshape(1, 3))


def kernel(x, edge_index, B_fourier, W_enc1, b_enc1, W_enc2, b_enc2,
           sage_Wl, sage_bl, sage_Wr, ln_g, ln_b, W_h1, b_h1, W_h2, b_h2):
    src = edge_index[0].astype(jnp.int32)
    dst = edge_index[1].astype(jnp.int32)
    pad = EPAD - E
    ar = jnp.arange(pad, dtype=jnp.int32)
    src_p = jnp.concatenate([src, (ar * 37) % N]).reshape(NCHUNK, CHUNK)
    dst_p = jnp.concatenate([dst, N + (ar % (NACC - N))]).reshape(NCHUNK, CHUNK)
    z128 = jnp.zeros((CHUNK, H), _f32)
    ones128 = jnp.ones((CHUNK, H), _f32)

    proj = _TWO_PI * (x[:, :3] @ B_fourier)
    h = _encoder(x, jnp.sin(proj), jnp.cos(proj), W_enc1, b_enc1, W_enc2,
                 b_enc2)

    degs = _get_deg()(dst_p, z128, ones128)
    d0 = degs[0, :, :16]
    d1 = degs[1, :, :16]
    for i in range(L):
        aggr = _get_agg()(h, src_p, dst_p, z128)
        h = _layer(h, aggr[0], aggr[1], d0, d1,
                   sage_Wl[i], sage_bl[i], sage_Wr[i], ln_g[i], ln_b[i])

    return _head(h, W_h1, b_h1, W_h2, b_h2)
